# Initial kernel scaffold; baseline (speedup 1.0000x reference)
import jax
import jax.numpy as jnp
from jax import lax
from jax.experimental import pallas as pl
from jax.experimental.pallas import tpu as pltpu
from jax.experimental.pallas import tpu_sc as plsc

B, L = 4096, 200
V, T, A = 100000, 26, 52
D = 64

NC, NS, LANES = 2, 16, 16
NW = NC * NS
N = B * L
PER_W = N // NW
C = 128
N_CHUNKS = PER_W // C


def _body(ids_hbm, tts_hbm, adms_hbm, code_hbm, type_hbm, adm_hbm, out_hbm,
          ids_v, tts_v, adms_v, rows_c, rows_t, rows_a, sem1, sem2, sem3):
    wid = lax.axis_index("s") * NC + lax.axis_index("c")

    pltpu.sync_copy(ids_hbm.at[wid], ids_v)
    pltpu.sync_copy(tts_hbm.at[wid], tts_v)
    pltpu.sync_copy(adms_hbm.at[wid], adms_v)

    def chunk(g, carry):
        cp1 = pltpu.async_copy(code_hbm.at[ids_v.at[g]], rows_c, sem1)
        cp2 = pltpu.async_copy(type_hbm.at[tts_v.at[g]], rows_t, sem2)
        cp3 = pltpu.async_copy(adm_hbm.at[adms_v.at[g]], rows_a, sem3)
        cp1.wait()
        cp2.wait()
        cp3.wait()

        def tok(t, c2):
            for col in range(D // LANES):
                s = pl.ds(col * LANES, LANES)
                rows_c[t, s] = rows_c[t, s] + rows_t[t, s] + rows_a[t, s]
            return c2

        lax.fori_loop(0, C, tok, 0)
        pltpu.sync_copy(rows_c, out_hbm.at[wid, g])
        return carry

    lax.fori_loop(0, N_CHUNKS, chunk, 0)


@jax.jit
def kernel(input_ids, token_types, adm_index, code_embed, type_embed,
           adm_embed):
    ids3 = input_ids.reshape(NW, N_CHUNKS, C)
    tts3 = token_types.reshape(NW, N_CHUNKS, C)
    adms3 = adm_index.reshape(NW, N_CHUNKS, C)

    mesh = plsc.VectorSubcoreMesh(core_axis_name="c", subcore_axis_name="s")
    out = pl.kernel(
        _body,
        out_type=jax.ShapeDtypeStruct((NW, N_CHUNKS, C, D), jnp.float32),
        mesh=mesh,
        scratch_types=[
            pltpu.VMEM((N_CHUNKS, C), jnp.int32),
            pltpu.VMEM((N_CHUNKS, C), jnp.int32),
            pltpu.VMEM((N_CHUNKS, C), jnp.int32),
            pltpu.VMEM((C, D), jnp.float32),
            pltpu.VMEM((C, D), jnp.float32),
            pltpu.VMEM((C, D), jnp.float32),
            pltpu.SemaphoreType.DMA,
            pltpu.SemaphoreType.DMA,
            pltpu.SemaphoreType.DMA,
        ],
    )(ids3, tts3, adms3, code_embed, type_embed, adm_embed)
    return out.reshape(B, L, D)


# speedup vs baseline: 3.8386x; 3.8386x over previous
import jax
import jax.numpy as jnp
from jax import lax
from jax.experimental import pallas as pl
from jax.experimental.pallas import tpu as pltpu
from jax.experimental.pallas import tpu_sc as plsc

B, L = 4096, 200
V, T, A = 100000, 26, 52
D = 64

NC, NS, LANES = 2, 16, 16
NW = NC * NS
N = B * L
PER_W = N // NW
C = 128
N_CHUNKS = PER_W // C


def _body(ids_hbm, tts_hbm, adms_hbm, code_hbm, type_hbm, adm_hbm, out_hbm,
          ids_v, tts_v, adms_v, rows_c, rows_t, rows_a, sem1, sem2, sem3):
    wid = lax.axis_index("s") * NC + lax.axis_index("c")

    pltpu.sync_copy(ids_hbm.at[wid], ids_v)
    pltpu.sync_copy(tts_hbm.at[wid], tts_v)
    pltpu.sync_copy(adms_hbm.at[wid], adms_v)

    def chunk(g, carry):
        cp1 = pltpu.async_copy(code_hbm.at[ids_v.at[g]], rows_c, sem1)
        cp2 = pltpu.async_copy(type_hbm.at[tts_v.at[g]], rows_t, sem2)
        cp3 = pltpu.async_copy(adm_hbm.at[adms_v.at[g]], rows_a, sem3)
        cp1.wait()
        cp2.wait()
        cp3.wait()

        def tok(t, c2):
            for col in range(D // LANES):
                s = pl.ds(col * LANES, LANES)
                rows_c[t, s] = rows_c[t, s] + rows_t[t, s] + rows_a[t, s]
            return c2

        lax.fori_loop(0, C, tok, 0)
        pltpu.sync_copy(rows_c, out_hbm.at[wid, g])
        return carry

    lax.fori_loop(0, N_CHUNKS, chunk, 0)


@jax.jit
def kernel(input_ids, token_types, adm_index, code_embed, type_embed,
           adm_embed):
    ids3 = input_ids.reshape(NW, N_CHUNKS, C)
    tts3 = token_types.reshape(NW, N_CHUNKS, C)
    adms3 = adm_index.reshape(NW, N_CHUNKS, C)

    mesh = plsc.VectorSubcoreMesh(core_axis_name="c", subcore_axis_name="s")
    out = pl.kernel(
        _body,
        out_type=jax.ShapeDtypeStruct((NW, N_CHUNKS, C, D), jnp.float32),
        mesh=mesh,
        compiler_params=pltpu.CompilerParams(use_tc_tiling_on_sc=False),
        scratch_types=[
            pltpu.VMEM((N_CHUNKS, C), jnp.int32),
            pltpu.VMEM((N_CHUNKS, C), jnp.int32),
            pltpu.VMEM((N_CHUNKS, C), jnp.int32),
            pltpu.VMEM((C, D), jnp.float32),
            pltpu.VMEM((C, D), jnp.float32),
            pltpu.VMEM((C, D), jnp.float32),
            pltpu.SemaphoreType.DMA,
            pltpu.SemaphoreType.DMA,
            pltpu.SemaphoreType.DMA,
        ],
    )(ids3, tts3, adms3, code_embed, type_embed, adm_embed)
    return out.reshape(B, L, D)
